# baseline (speedup 1.0000x reference)
import functools

import jax
import jax.numpy as jnp
from jax import lax
from jax.experimental import pallas as pl
from jax.experimental.pallas import tpu as pltpu
from jax.experimental.pallas import tpu_sc as plsc

N = 10000
E = 320000
D = 128
NC = 2
NS = 16
NW = NC * NS
K = 128
NP = 10016
EP = NW * 80 * K
CPW = EP // (NW * K)
ES = EP // K
RPS = 624
TAIL = NP - NS * RPS

_sc_mesh = plsc.VectorSubcoreMesh(core_axis_name="c", subcore_axis_name="s")


def _deg_body(dst_hbm, ones_hbm, out_hbm, dsta, onesv, deg_sh, ss):
    c = lax.axis_index("c")
    s = lax.axis_index("s")
    wid = s * NC + c
    r0 = s * RPS
    pltpu.sync_copy(ones_hbm.at[pl.ds(r0, RPS)], deg_sh.at[pl.ds(r0, RPS)])

    @pl.when(s == 0)
    def _():
        pltpu.sync_copy(ones_hbm.at[pl.ds(NS * RPS, TAIL)],
                        deg_sh.at[pl.ds(NS * RPS, TAIL)])

    pltpu.sync_copy(ones_hbm.at[pl.ds(0, K)], onesv)
    pltpu.sync_copy(dst_hbm.at[pl.ds(wid * CPW, CPW)], dsta)
    plsc.subcore_barrier()

    def body(j, carry):
        pltpu.async_copy(onesv, deg_sh.at[dsta.at[j]], ss, add=True)

        @pl.when(j > 0)
        def _():
            pltpu.make_async_copy(onesv, deg_sh.at[dsta.at[0]], ss).wait()

        return carry

    lax.fori_loop(0, CPW, body, 0)
    pltpu.make_async_copy(onesv, deg_sh.at[dsta.at[0]], ss).wait()
    plsc.subcore_barrier()
    pltpu.sync_copy(deg_sh.at[pl.ds(r0, RPS)],
                    out_hbm.at[pl.ds(c * NP + r0, RPS)])

    @pl.when(s == 0)
    def _():
        pltpu.sync_copy(deg_sh.at[pl.ds(NS * RPS, TAIL)],
                        out_hbm.at[pl.ds(c * NP + NS * RPS, TAIL)])


_deg_call = functools.partial(
    pl.kernel,
    out_type=jax.ShapeDtypeStruct((NC * NP, D), jnp.float32),
    mesh=_sc_mesh,
    scratch_types=[
        pltpu.VMEM((CPW, K), jnp.int32),
        pltpu.VMEM((K, D), jnp.float32),
        pltpu.VMEM_SHARED((NP, D), jnp.float32),
        pltpu.SemaphoreType.DMA,
    ],
)(_deg_body)


def _agg_body(table_hbm, src_hbm, dst_hbm, out_hbm, srcv, dstv, rows,
              acc_sh, isr, ids, gs, ss):
    c = lax.axis_index("c")
    s = lax.axis_index("s")
    wid = s * NC + c
    r0 = s * RPS
    pltpu.sync_copy(table_hbm.at[pl.ds(r0, RPS)], acc_sh.at[pl.ds(r0, RPS)])

    @pl.when(s == 0)
    def _():
        pltpu.sync_copy(table_hbm.at[pl.ds(NS * RPS, TAIL)],
                        acc_sh.at[pl.ds(NS * RPS, TAIL)])

    plsc.subcore_barrier()
    base0 = wid * CPW * K

    def isrc_start(j, a):
        pltpu.async_copy(src_hbm.at[pl.ds(base0 + j * K, K)], srcv[a], isr[a])

    def isrc_wait(a):
        pltpu.make_async_copy(src_hbm.at[pl.ds(0, K)], srcv[a], isr[a]).wait()

    def idst_start(j, a):
        pltpu.async_copy(dst_hbm.at[pl.ds(base0 + j * K, K)], dstv[a], ids[a])

    def idst_wait(a):
        pltpu.make_async_copy(dst_hbm.at[pl.ds(0, K)], dstv[a], ids[a]).wait()

    def g_start(a):
        pltpu.async_copy(table_hbm.at[srcv[a]], rows[a], gs[a])

    def g_wait(a):
        pltpu.make_async_copy(table_hbm.at[srcv[a]], rows[a], gs[a]).wait()

    def s_start(a):
        pltpu.async_copy(rows[a], acc_sh.at[dstv[a]], ss[a], add=True)

    def s_wait(a):
        pltpu.make_async_copy(rows[a], acc_sh.at[dstv[a]], ss[a]).wait()

    isrc_start(0, 0)
    idst_start(0, 0)
    isrc_start(1, 1)
    isrc_wait(0)
    g_start(0)

    def body(t, carry):
        j0 = 2 * t
        more = t + 1 < CPW // 2
        g_wait(0)
        idst_wait(0)
        s_start(0)

        @pl.when(t > 0)
        def _():
            s_wait(1)

        idst_start(j0 + 1, 1)
        isrc_wait(1)
        g_start(1)

        @pl.when(more)
        def _():
            isrc_start(j0 + 2, 0)

        g_wait(1)
        idst_wait(1)
        s_start(1)
        s_wait(0)

        @pl.when(more)
        def _():
            idst_start(j0 + 2, 0)
            isrc_wait(0)
            g_start(0)
            isrc_start(j0 + 3, 1)

        return carry

    lax.fori_loop(0, CPW // 2, body, 0)
    s_wait(1)
    plsc.subcore_barrier()
    pltpu.sync_copy(acc_sh.at[pl.ds(r0, RPS)],
                    out_hbm.at[pl.ds(c * NP + r0, RPS)])

    @pl.when(s == 0)
    def _():
        pltpu.sync_copy(acc_sh.at[pl.ds(NS * RPS, TAIL)],
                        out_hbm.at[pl.ds(c * NP + NS * RPS, TAIL)])


_agg_call = functools.partial(
    pl.kernel,
    out_type=jax.ShapeDtypeStruct((NC * NP, D), jnp.float32),
    mesh=_sc_mesh,
    scratch_types=[
        [pltpu.VMEM((K,), jnp.int32), pltpu.VMEM((K,), jnp.int32)],
        [pltpu.VMEM((K,), jnp.int32), pltpu.VMEM((K,), jnp.int32)],
        [pltpu.VMEM((K, D), jnp.float32), pltpu.VMEM((K, D), jnp.float32)],
        pltpu.VMEM_SHARED((NP, D), jnp.float32),
        [pltpu.SemaphoreType.DMA, pltpu.SemaphoreType.DMA],
        [pltpu.SemaphoreType.DMA, pltpu.SemaphoreType.DMA],
        [pltpu.SemaphoreType.DMA, pltpu.SemaphoreType.DMA],
        [pltpu.SemaphoreType.DMA, pltpu.SemaphoreType.DMA],
    ],
)(_agg_body)


BR = 1000

def _dis(d0_ref, d1_ref):
    deg = d0_ref[:, 0:1] + d1_ref[:, 0:1] - 1.0
    return lax.rsqrt(deg)


def _tc1_body(x_ref, w_ref, d0_ref, d1_ref, o_ref):
    h = jnp.dot(x_ref[...], w_ref[...], preferred_element_type=jnp.float32)
    o_ref[...] = h * _dis(d0_ref, d1_ref)


def _tc2_body(p0_ref, p1_ref, t_ref, d0_ref, d1_ref, b_ref, w_ref, o_ref):
    dis = _dis(d0_ref, d1_ref)
    z = dis * (p0_ref[...] + p1_ref[...] - t_ref[...]) + b_ref[...]
    z = jnp.maximum(z, 0.0)
    o_ref[...] = jnp.dot(z, w_ref[...],
                         preferred_element_type=jnp.float32) * dis


def _tc3_body(q0_ref, q1_ref, t_ref, d0_ref, d1_ref, b_ref, o_ref):
    dis = _dis(d0_ref, d1_ref)
    o_ref[...] = dis * (q0_ref[...] + q1_ref[...] - t_ref[...]) + b_ref[...]


_row = pl.BlockSpec((BR, D), lambda i: (i, 0))
_deg_blk = pl.BlockSpec((BR, D), lambda i: (i, 0))
_wfull = pl.BlockSpec((D, D), lambda i: (0, 0))
_bfull = pl.BlockSpec((1, D), lambda i: (0, 0))
_grid = (N // BR,)
_out_rows = jax.ShapeDtypeStruct((N, D), jnp.float32)

_tc1 = pl.pallas_call(
    _tc1_body, grid=_grid,
    in_specs=[_row, _wfull, _deg_blk, _deg_blk],
    out_specs=_row, out_shape=_out_rows)

_tc2 = pl.pallas_call(
    _tc2_body, grid=_grid,
    in_specs=[_row, _row, _row, _deg_blk, _deg_blk, _bfull, _wfull],
    out_specs=_row, out_shape=_out_rows)

_tc3 = pl.pallas_call(
    _tc3_body, grid=_grid,
    in_specs=[_row, _row, _row, _deg_blk, _deg_blk, _bfull],
    out_specs=_row, out_shape=_out_rows)


@jax.jit
def kernel(x, edge_index, W1, b1, W2, b2):
    src = edge_index[0].astype(jnp.int32)
    dst = edge_index[1].astype(jnp.int32)
    pad = jnp.full((EP - E,), N, jnp.int32)
    src1 = jnp.concatenate([src, pad])
    dst1 = jnp.concatenate([dst, pad])
    dst2 = dst1.reshape(ES, K)
    ones = jnp.ones((NP, D), jnp.float32)

    degp = _deg_call(dst2, ones)
    d0, d1 = degp[:N], degp[NP:NP + N]

    b1r = b1.reshape(1, D)
    b2r = b2.reshape(1, D)

    h1p = _tc1(x, W1, d0, d1)
    h1pp = jnp.pad(h1p, ((0, NP - N), (0, 0)))
    p = _agg_call(h1pp, src1, dst1)
    h2p = _tc2(p[:N], p[NP:NP + N], h1p, d0, d1, b1r, W2)
    h2pp = jnp.pad(h2p, ((0, NP - N), (0, 0)))
    q = _agg_call(h2pp, src1, dst1)
    return _tc3(q[:N], q[NP:NP + N], h2p, d0, d1, b2r)


# speedup vs baseline: 3.5738x; 3.5738x over previous
import functools

import jax
import jax.numpy as jnp
from jax import lax
from jax.experimental import pallas as pl
from jax.experimental.pallas import tpu as pltpu
from jax.experimental.pallas import tpu_sc as plsc

N = 10000
E = 320000
D = 128
NC = 2
NS = 16
NW = NC * NS
K = 128
NP = 10240
EP = NW * 80 * K
CPW = EP // (NW * K)
ES = EP // K
RPS = NP // NS

_sc_mesh = plsc.VectorSubcoreMesh(core_axis_name="c", subcore_axis_name="s")


def _deg_body(dst_hbm, out_hbm, dsta, onesv, deg_sh, ss):
    c = lax.axis_index("c")
    s = lax.axis_index("s")
    wid = s * NC + c
    r0 = s * RPS

    for j in range(K // 16):
        onesv[pl.ds(16 * j, 16)] = jnp.zeros((16,), jnp.float32)
    for t in range(RPS // K):
        pltpu.sync_copy(onesv, deg_sh.at[pl.ds(r0 + t * K, K)])
    for j in range(K // 16):
        onesv[pl.ds(16 * j, 16)] = jnp.full((16,), 1.0, jnp.float32)
    pltpu.sync_copy(dst_hbm.at[pl.ds(wid * CPW, CPW)], dsta)
    plsc.subcore_barrier()

    def body(j, carry):
        pltpu.async_copy(onesv, deg_sh.at[dsta.at[j]], ss, add=True)

        @pl.when(j > 0)
        def _():
            pltpu.make_async_copy(onesv, deg_sh.at[dsta.at[0]], ss).wait()

        return carry

    lax.fori_loop(0, CPW, body, 0)
    pltpu.make_async_copy(onesv, deg_sh.at[dsta.at[0]], ss).wait()
    plsc.subcore_barrier()
    pltpu.sync_copy(deg_sh.at[pl.ds(r0, RPS)],
                    out_hbm.at[pl.ds(c * NP + r0, RPS)])


_deg_call = functools.partial(
    pl.kernel,
    out_type=jax.ShapeDtypeStruct((NC * NP,), jnp.float32),
    mesh=_sc_mesh,
    scratch_types=[
        pltpu.VMEM((CPW, K), jnp.int32),
        pltpu.VMEM((K,), jnp.float32),
        pltpu.VMEM_SHARED((NP,), jnp.float32),
        pltpu.SemaphoreType.DMA,
    ],
)(_deg_body)


def _agg_body(table_hbm, src_hbm, dst_hbm, out_hbm, srcv, dstv, rows,
              acc_sh, isr, ids, gs, ss):
    c = lax.axis_index("c")
    s = lax.axis_index("s")
    wid = s * NC + c
    r0 = s * RPS
    base0 = wid * CPW * K

    def isrc_start(j, a):
        pltpu.async_copy(src_hbm.at[pl.ds(base0 + j * K, K)], srcv[a], isr[a])

    def isrc_wait(a):
        pltpu.make_async_copy(src_hbm.at[pl.ds(0, K)], srcv[a], isr[a]).wait()

    def idst_start(j, a):
        pltpu.async_copy(dst_hbm.at[pl.ds(base0 + j * K, K)], dstv[a], ids[a])

    def idst_wait(a):
        pltpu.make_async_copy(dst_hbm.at[pl.ds(0, K)], dstv[a], ids[a]).wait()

    def g_start(a):
        pltpu.async_copy(table_hbm.at[srcv[a]], rows[a], gs[a])

    def g_wait(a):
        pltpu.make_async_copy(table_hbm.at[srcv[a]], rows[a], gs[a]).wait()

    def s_start(a):
        pltpu.async_copy(rows[a], acc_sh.at[dstv[a]], ss[a], add=True)

    def s_wait(a):
        pltpu.make_async_copy(rows[a], acc_sh.at[dstv[a]], ss[a]).wait()

    isrc_start(0, 0)
    idst_start(0, 0)
    isrc_start(1, 1)
    pltpu.sync_copy(table_hbm.at[pl.ds(r0, RPS)], acc_sh.at[pl.ds(r0, RPS)])
    isrc_wait(0)
    g_start(0)
    plsc.subcore_barrier()

    def body(t, carry):
        j0 = 2 * t
        more = t + 1 < CPW // 2
        g_wait(0)
        idst_wait(0)
        s_start(0)

        @pl.when(t > 0)
        def _():
            s_wait(1)

        idst_start(j0 + 1, 1)
        isrc_wait(1)
        g_start(1)

        @pl.when(more)
        def _():
            isrc_start(j0 + 2, 0)

        g_wait(1)
        idst_wait(1)
        s_start(1)
        s_wait(0)

        @pl.when(more)
        def _():
            idst_start(j0 + 2, 0)
            isrc_wait(0)
            g_start(0)
            isrc_start(j0 + 3, 1)

        return carry

    lax.fori_loop(0, CPW // 2, body, 0)
    s_wait(1)
    plsc.subcore_barrier()
    pltpu.sync_copy(acc_sh.at[pl.ds(r0, RPS)],
                    out_hbm.at[pl.ds(c * NP + r0, RPS)])


_agg_call = functools.partial(
    pl.kernel,
    out_type=jax.ShapeDtypeStruct((NC * NP, D), jnp.float32),
    mesh=_sc_mesh,
    scratch_types=[
        [pltpu.VMEM((K,), jnp.int32), pltpu.VMEM((K,), jnp.int32)],
        [pltpu.VMEM((K,), jnp.int32), pltpu.VMEM((K,), jnp.int32)],
        [pltpu.VMEM((K, D), jnp.float32), pltpu.VMEM((K, D), jnp.float32)],
        pltpu.VMEM_SHARED((NP, D), jnp.float32),
        [pltpu.SemaphoreType.DMA, pltpu.SemaphoreType.DMA],
        [pltpu.SemaphoreType.DMA, pltpu.SemaphoreType.DMA],
        [pltpu.SemaphoreType.DMA, pltpu.SemaphoreType.DMA],
        [pltpu.SemaphoreType.DMA, pltpu.SemaphoreType.DMA],
    ],
)(_agg_body)


BR = 2000

def _tc1_body(x_ref, w_ref, d_ref, o_ref):
    h = jnp.dot(x_ref[...], w_ref[...], preferred_element_type=jnp.float32)
    o_ref[...] = h * lax.rsqrt(d_ref[...])


def _tc2_body(p0_ref, p1_ref, t_ref, d_ref, b_ref, w_ref, o_ref):
    dis = lax.rsqrt(d_ref[...])
    z = dis * (p0_ref[...] + p1_ref[...] - t_ref[...]) + b_ref[...]
    z = jnp.maximum(z, 0.0)
    o_ref[...] = jnp.dot(z, w_ref[...],
                         preferred_element_type=jnp.float32) * dis


def _tc3_body(q0_ref, q1_ref, t_ref, d_ref, b_ref, o_ref):
    dis = lax.rsqrt(d_ref[...])
    o_ref[...] = dis * (q0_ref[...] + q1_ref[...] - t_ref[...]) + b_ref[...]


_row = pl.BlockSpec((BR, D), lambda i: (i, 0))
_wfull = pl.BlockSpec((D, D), lambda i: (0, 0))
_bfull = pl.BlockSpec((1, D), lambda i: (0, 0))
_grid = (N // BR,)
_out_rows = jax.ShapeDtypeStruct((N, D), jnp.float32)

_tc1 = pl.pallas_call(
    _tc1_body, grid=_grid,
    in_specs=[_row, _wfull, _row],
    out_specs=_row, out_shape=_out_rows)

_tc2 = pl.pallas_call(
    _tc2_body, grid=_grid,
    in_specs=[_row, _row, _row, _row, _bfull, _wfull],
    out_specs=_row, out_shape=_out_rows)

_tc3 = pl.pallas_call(
    _tc3_body, grid=_grid,
    in_specs=[_row, _row, _row, _row, _bfull],
    out_specs=_row, out_shape=_out_rows)


@jax.jit
def kernel(x, edge_index, W1, b1, W2, b2):
    src = edge_index[0].astype(jnp.int32)
    dst = edge_index[1].astype(jnp.int32)
    pad = N + jnp.arange(EP - E, dtype=jnp.int32) % (NP - N)
    src1 = jnp.concatenate([src, pad])
    dst1 = jnp.concatenate([dst, pad])
    dst2 = dst1.reshape(ES, K)

    degp = _deg_call(dst2)
    deg = degp[:N] + degp[NP:NP + N] + 1.0
    degb = jnp.broadcast_to(deg[:, None], (N, D))

    b1r = b1.reshape(1, D)
    b2r = b2.reshape(1, D)

    h1p = _tc1(x, W1, degb)
    h1pp = jnp.pad(h1p, ((0, NP - N), (0, 0)))
    p = _agg_call(h1pp, src1, dst1)
    h2p = _tc2(p[:N], p[NP:NP + N], h1p, degb, b1r, W2)
    h2pp = jnp.pad(h2p, ((0, NP - N), (0, 0)))
    q = _agg_call(h2pp, src1, dst1)
    return _tc3(q[:N], q[NP:NP + N], h2p, degb, b2r)
